# Initial kernel scaffold; baseline (speedup 1.0000x reference)
import jax
import jax.numpy as jnp
from jax.experimental import pallas as pl


def kernel(x, edge_index, batch, W1, b1, W2, b2, Wg1, bg1, Wg2, bg2):
    raise NotImplementedError("write your pallas kernel here")


# speedup vs baseline: 5.9578x; 5.9578x over previous
import functools

import jax
import jax.numpy as jnp
from jax import lax
from jax.experimental import pallas as pl
from jax.experimental.pallas import tpu as pltpu
from jax.experimental.pallas import tpu_sc as plsc

N_NODES = 10000
N_EDGES = 320000
N_GRAPHS = 256

NC, NS = 2, 16
NROW = 10240
PAD_ROW = 10016
CHUNK = 128
CH_ALL = 160
CH_HALF = CH_ALL // 2
NPAD = NS * CH_ALL * CHUNK
ROWS_S = NROW // NS
W = 128
DW = 128
BLK = 512


def _mesh():
    return plsc.VectorSubcoreMesh(
        core_axis_name="c", subcore_axis_name="s", num_cores=NC, num_subcores=NS
    )


RANGE = NROW // NC
RACC = RANGE + 128
RZERO = RACC // NS
ROUT = RANGE // NS


def _sc_degree_body(dst_hbm, ones_hbm, zeros_hbm, out_hbm, dst_v, ones_v, acc):
    c = lax.axis_index("c")
    s = lax.axis_index("s")
    pltpu.sync_copy(ones_hbm, ones_v)
    pltpu.sync_copy(zeros_hbm, acc.at[pl.ds(s * RZERO, RZERO)])
    pltpu.sync_copy(dst_hbm.at[s], dst_v)
    base = c * RANGE

    @pl.loop(0, CH_ALL)
    def _(k):
        for j in range(CHUNK // 16):
            sl = pl.ds(j * 16, 16)
            t = dst_v[k, sl] - base
            valid = (t >= 0) & (t < RANGE)
            dst_v[k, sl] = jnp.where(valid, t, RANGE)

    plsc.subcore_barrier()

    @pl.loop(0, CH_ALL)
    def _(k):
        pltpu.sync_copy(ones_v, acc.at[dst_v.at[k]], add=True)

    plsc.subcore_barrier()
    pltpu.sync_copy(
        acc.at[pl.ds(s * ROUT, ROUT)],
        out_hbm.at[pl.ds(c * RANGE + s * ROUT, ROUT)],
    )


@functools.cache
def _get_sc_degree():
    return functools.partial(
        pl.kernel,
        out_type=jax.ShapeDtypeStruct((NROW, DW), jnp.float32),
        mesh=_mesh(),
        scratch_types=[
            pltpu.VMEM((CH_ALL, CHUNK), jnp.int32),
            pltpu.VMEM((CHUNK, DW), jnp.float32),
            pltpu.VMEM_SHARED((RACC, DW), jnp.float32),
        ],
    )(_sc_degree_body)


def _sc_edges_body(
    src_hbm, dst_hbm, table_hbm, zeros_hbm, out_hbm,
    src_v, dst_v, r0, r1, acc, sem0, sem1,
):
    c = lax.axis_index("c")
    s = lax.axis_index("s")
    pltpu.sync_copy(zeros_hbm, acc.at[pl.ds(s * RZERO, RZERO)])
    pltpu.sync_copy(src_hbm.at[s], src_v)
    pltpu.sync_copy(dst_hbm.at[s], dst_v)
    base = c * RANGE

    @pl.loop(0, CH_ALL)
    def _(k):
        for j in range(CHUNK // 16):
            sl = pl.ds(j * 16, 16)
            t = dst_v[k, sl] - base
            valid = (t >= 0) & (t < RANGE)
            dst_v[k, sl] = jnp.where(valid, t, RANGE)

    plsc.subcore_barrier()

    pltpu.async_copy(table_hbm.at[src_v.at[0]], r0, sem0)

    @pl.loop(0, CH_ALL, step=2)
    def _(k):
        pltpu.async_copy(table_hbm.at[src_v.at[k + 1]], r1, sem1)
        pltpu.make_async_copy(table_hbm.at[src_v.at[k]], r0, sem0).wait()
        pltpu.sync_copy(r0, acc.at[dst_v.at[k]], add=True)

        @pl.when(k + 2 < CH_ALL)
        def _():
            pltpu.async_copy(table_hbm.at[src_v.at[k + 2]], r0, sem0)

        pltpu.make_async_copy(table_hbm.at[src_v.at[k + 1]], r1, sem1).wait()
        pltpu.sync_copy(r1, acc.at[dst_v.at[k + 1]], add=True)

    plsc.subcore_barrier()
    pltpu.sync_copy(
        acc.at[pl.ds(s * ROUT, ROUT)],
        out_hbm.at[pl.ds(c * RANGE + s * ROUT, ROUT)],
    )


@functools.cache
def _get_sc_edges():
    return functools.partial(
        pl.kernel,
        out_type=jax.ShapeDtypeStruct((NROW, W), jnp.float32),
        mesh=_mesh(),
        scratch_types=[
            pltpu.VMEM((CH_ALL, CHUNK), jnp.int32),
            pltpu.VMEM((CH_ALL, CHUNK), jnp.int32),
            pltpu.VMEM((CHUNK, W), jnp.float32),
            pltpu.VMEM((CHUNK, W), jnp.float32),
            pltpu.VMEM_SHARED((RACC, W), jnp.float32),
            pltpu.SemaphoreType.DMA,
            pltpu.SemaphoreType.DMA,
        ],
    )(_sc_edges_body)


def _dinv_of(degp_ref):
    deg = degp_ref[:, :1] + 1.0
    return lax.rsqrt(deg)


def _tc_scale_body(x_ref, degp_ref, u_ref):
    u_ref[:] = x_ref[:] * _dinv_of(degp_ref)


def _tc_scale(x_pad, degp):
    return pl.pallas_call(
        _tc_scale_body,
        grid=(NROW // BLK,),
        in_specs=[
            pl.BlockSpec((BLK, 128), lambda r: (r, 0)),
            pl.BlockSpec((BLK, DW), lambda r: (r, 0)),
        ],
        out_specs=pl.BlockSpec((BLK, 128), lambda r: (r, 0)),
        out_shape=jax.ShapeDtypeStruct((NROW, 128), jnp.float32),
    )(x_pad, degp)


def _tc_layer1_body(t_ref, u_ref, degp_ref, w1_ref, b1_ref, u2_ref):
    dinv = _dinv_of(degp_ref)
    v = (t_ref[:] + u_ref[:]) * dinv
    h1 = jnp.maximum(
        jnp.dot(v, w1_ref[:], preferred_element_type=jnp.float32)
        + b1_ref[:][None, :],
        0.0,
    )
    u2_ref[:] = h1 * dinv


def _tc_layer1(t1p, u1, degp, w1, b1):
    return pl.pallas_call(
        _tc_layer1_body,
        grid=(NROW // BLK,),
        in_specs=[
            pl.BlockSpec((BLK, 128), lambda r: (r, 0)),
            pl.BlockSpec((BLK, 128), lambda r: (r, 0)),
            pl.BlockSpec((BLK, DW), lambda r: (r, 0)),
            pl.BlockSpec((128, 128), lambda r: (0, 0)),
            pl.BlockSpec((128,), lambda r: (0,)),
        ],
        out_specs=pl.BlockSpec((BLK, 128), lambda r: (r, 0)),
        out_shape=jax.ShapeDtypeStruct((NROW, 128), jnp.float32),
    )(t1p, u1, degp, w1, b1)


def _tc_final_body(
    t_ref, u_ref, degp_ref, w2_ref, b2_ref, batch_ref, wg1_ref, bg1_ref,
    wg2_ref, bg2_ref, out_ref, psum, cnt,
):
    r = pl.program_id(0)

    @pl.when(r == 0)
    def _():
        psum[:] = jnp.zeros_like(psum)
        cnt[:] = jnp.zeros_like(cnt)

    dinv = _dinv_of(degp_ref)
    v = (t_ref[:] + u_ref[:]) * dinv
    h2 = jnp.maximum(
        jnp.dot(v, w2_ref[:], preferred_element_type=jnp.float32)
        + b2_ref[:][None, :],
        0.0,
    )
    oh = (
        batch_ref[:][:, None]
        == lax.broadcasted_iota(jnp.int32, (1, N_GRAPHS), 1)
    ).astype(jnp.float32)
    psum[:] += lax.dot_general(
        oh, h2, (((0,), (0,)), ((), ())), preferred_element_type=jnp.float32
    )
    cnt[:] += jnp.sum(oh, axis=0, keepdims=True)

    @pl.when(r == pl.num_programs(0) - 1)
    def _():
        g = psum[:] / jnp.maximum(cnt[:], 1.0).reshape(N_GRAPHS, 1)
        t = jnp.maximum(
            jnp.dot(g, wg1_ref[:], preferred_element_type=jnp.float32)
            + bg1_ref[:][None, :],
            0.0,
        )
        out_ref[:] = (
            jnp.dot(t, wg2_ref[:], preferred_element_type=jnp.float32)
            + bg2_ref[:][None, :]
        )


def _tc_final(t2p, u2, degp, w2, b2, batch_pad, wg1, bg1, wg2, bg2):
    return pl.pallas_call(
        _tc_final_body,
        grid=(NROW // BLK,),
        in_specs=[
            pl.BlockSpec((BLK, 128), lambda r: (r, 0)),
            pl.BlockSpec((BLK, 128), lambda r: (r, 0)),
            pl.BlockSpec((BLK, DW), lambda r: (r, 0)),
            pl.BlockSpec((128, 256), lambda r: (0, 0)),
            pl.BlockSpec((256,), lambda r: (0,)),
            pl.BlockSpec((BLK,), lambda r: (r,)),
            pl.BlockSpec((256, 1024), lambda r: (0, 0)),
            pl.BlockSpec((1024,), lambda r: (0,)),
            pl.BlockSpec((1024, 128), lambda r: (0, 0)),
            pl.BlockSpec((128,), lambda r: (0,)),
        ],
        out_specs=pl.BlockSpec((N_GRAPHS, 128), lambda r: (0, 0)),
        out_shape=jax.ShapeDtypeStruct((N_GRAPHS, 128), jnp.float32),
        scratch_shapes=[
            pltpu.VMEM((N_GRAPHS, N_GRAPHS), jnp.float32),
            pltpu.VMEM((1, N_GRAPHS), jnp.float32),
        ],
    )(t2p, u2, degp, w2, b2, batch_pad, wg1, bg1, wg2, bg2)


def kernel(x, edge_index, batch, W1, b1, W2, b2, Wg1, bg1, Wg2, bg2):
    i32 = jnp.int32
    src = edge_index[0].astype(i32)
    dst = edge_index[1].astype(i32)
    pad = NPAD - N_EDGES
    src_r = jnp.concatenate([src, jnp.full((pad,), PAD_ROW, i32)]).reshape(
        NS, CH_ALL, CHUNK
    )
    dst_r = jnp.concatenate([dst, jnp.full((pad,), PAD_ROW, i32)]).reshape(
        NS, CH_ALL, CHUNK
    )

    x_pad = jnp.pad(x, ((0, NROW - N_NODES), (0, 0)))
    batch_pad = jnp.pad(
        batch.astype(i32), (0, NROW - N_NODES), constant_values=N_GRAPHS
    )

    ones_blk = jnp.ones((CHUNK, W), jnp.float32)
    zeros_blk = jnp.zeros((RZERO, W), jnp.float32)

    degp = _get_sc_degree()(dst_r, ones_blk, zeros_blk)
    u1 = _tc_scale(x_pad, degp)
    t1p = _get_sc_edges()(src_r, dst_r, u1, zeros_blk)
    u2 = _tc_layer1(t1p, u1, degp, W1, b1)
    t2p = _get_sc_edges()(src_r, dst_r, u2, zeros_blk)
    return _tc_final(t2p, u2, degp, W2, b2, batch_pad, Wg1, bg1, Wg2, bg2)
